# Initial kernel scaffold; baseline (speedup 1.0000x reference)
import jax
import jax.numpy as jnp
from jax.experimental import pallas as pl


def kernel(h, coords, edge_index, a, Wc1, bc1, Wc2, bc2, Wc3, We1, be1, We2, be2, Wa, ba, Wn1, bn1, Wn2, bn2):
    raise NotImplementedError("write your pallas kernel here")


# speedup vs baseline: 1.7841x; 1.7841x over previous
import functools

import jax
import jax.numpy as jnp
from jax import lax
from jax.experimental import pallas as pl
from jax.experimental.pallas import tpu as pltpu
from jax.experimental.pallas import tpu_sc as plsc

N = 10000
E = 320000
HID = 128
EDF = 16
D = 256
NC, NS = 2, 16
NW = NC * NS

_R = 2 * E
_RPW = _R // NW
_GC = 80
_GCHUNKS = _RPW // _GC


def _gather_body(table, idx, out, idx_v, rows_v, sem):
    c = lax.axis_index("c")
    s = lax.axis_index("s")
    wid = s * NC + c
    base = wid * _RPW

    @pl.loop(0, _GCHUNKS)
    def _(i):
        off = base + i * _GC
        pltpu.sync_copy(idx.at[pl.ds(off, _GC)], idx_v)
        pltpu.async_copy(table.at[idx_v], rows_v, sem).wait()
        pltpu.sync_copy(rows_v, out.at[pl.ds(off, _GC)])


@functools.cache
def _get_gather():
    return pl.kernel(
        _gather_body,
        out_type=jax.ShapeDtypeStruct((_R, D), jnp.float32),
        mesh=plsc.VectorSubcoreMesh(core_axis_name="c", subcore_axis_name="s",
                                    num_cores=NC, num_subcores=NS),
        scratch_types=[
            pltpu.VMEM((_GC,), jnp.int32),
            pltpu.VMEM((_GC, D), jnp.float32),
            pltpu.SemaphoreType.DMA,
        ],
    )


_EPT = E // NS
_SCC = 80
_SCHUNKS = _EPT // _SCC
_ZROWS = 632
_ZLAST = N - (NS - 1) * _ZROWS


def _scatter_body(msg, dst, z, out, idx_v, rows_v, acc, sem):
    c = lax.axis_index("c")
    s = lax.axis_index("s")
    r0 = s * _ZROWS

    @pl.when(s < NS - 1)
    def _():
        pltpu.sync_copy(z.at[pl.ds(r0, _ZROWS)], acc.at[pl.ds(r0, _ZROWS)])

    @pl.when(s == NS - 1)
    def _():
        pltpu.sync_copy(z.at[pl.ds(r0, _ZLAST)], acc.at[pl.ds(r0, _ZLAST)])

    plsc.subcore_barrier()

    base = s * _EPT
    col = c * HID

    @pl.loop(0, _SCHUNKS)
    def _(i):
        off = base + i * _SCC
        pltpu.sync_copy(dst.at[pl.ds(off, _SCC)], idx_v)
        pltpu.sync_copy(msg.at[pl.ds(off, _SCC), pl.ds(col, HID)], rows_v)
        pltpu.sync_copy(rows_v, acc.at[idx_v], add=True)

    plsc.subcore_barrier()

    @pl.when(s < NS - 1)
    def _():
        pltpu.sync_copy(acc.at[pl.ds(r0, _ZROWS)],
                        out.at[c, pl.ds(r0, _ZROWS)])

    @pl.when(s == NS - 1)
    def _():
        pltpu.sync_copy(acc.at[pl.ds(r0, _ZLAST)],
                        out.at[c, pl.ds(r0, _ZLAST)])


@functools.cache
def _get_scatter():
    return pl.kernel(
        _scatter_body,
        out_type=jax.ShapeDtypeStruct((NC, N, HID), jnp.float32),
        mesh=plsc.VectorSubcoreMesh(core_axis_name="c", subcore_axis_name="s",
                                    num_cores=NC, num_subcores=NS),
        scratch_types=[
            pltpu.VMEM((_SCC,), jnp.int32),
            pltpu.VMEM((_SCC, HID), jnp.float32),
            pltpu.VMEM_SHARED((N, HID), jnp.float32),
            pltpu.SemaphoreType.DMA,
        ],
    )


_BE = 1280
_NBLK = E // _BE
_PREC = jax.lax.Precision.HIGHEST


def _silu(x):
    return x * jax.nn.sigmoid(x)


def _edge_mlp_body(hs_ref, hd_ref, a_ref, W1s, W1d, w1r, W1a, b1,
                   Wc2r, bc2r, We2r, be2r, wc3, wa, ba2, out_ref):
    hs = hs_ref[:, :HID]
    hd = hd_ref[:, :HID]
    d16 = hs_ref[:, HID:HID + EDF] - hd_ref[:, HID:HID + EDF]
    rad = jnp.sum(d16 * d16, axis=1, keepdims=True)
    t = (jnp.dot(hs, W1s[:], precision=_PREC, preferred_element_type=jnp.float32)
         + jnp.dot(hd, W1d[:], precision=_PREC, preferred_element_type=jnp.float32)
         + jnp.dot(a_ref[:], W1a[:], precision=_PREC, preferred_element_type=jnp.float32)
         + rad * w1r[:] + b1[:])
    t = _silu(t)
    c2 = _silu(jnp.dot(t[:, :HID], Wc2r[:], precision=_PREC,
                       preferred_element_type=jnp.float32) + bc2r[:])
    m2 = _silu(jnp.dot(t[:, HID:], We2r[:], precision=_PREC,
                       preferred_element_type=jnp.float32) + be2r[:])
    scale = jnp.sum(c2 * wc3[:], axis=1, keepdims=True)
    att = jax.nn.sigmoid(jnp.sum(m2 * wa[:], axis=1, keepdims=True) + ba2[:])
    out_ref[:, :HID] = att * m2
    out_ref[:, HID:] = jnp.concatenate(
        [(scale / (rad + 1.0)) * d16,
         jnp.zeros((_BE, D - HID - EDF), jnp.float32)], axis=1)


def _edge_mlp(gathered, a, W1s, W1d, w1r, W1a, b1, Wc2, bc2, We2, be2,
              wc3, wa, ba2):
    wfull = lambda shape: pl.BlockSpec(shape, lambda i: (0, 0))
    return pl.pallas_call(
        _edge_mlp_body,
        grid=(_NBLK,),
        in_specs=[
            pl.BlockSpec((_BE, D), lambda i: (i, 0)),
            pl.BlockSpec((_BE, D), lambda i: (i + _NBLK, 0)),
            pl.BlockSpec((_BE, EDF), lambda i: (i, 0)),
            wfull((HID, 2 * HID)), wfull((HID, 2 * HID)), wfull((1, 2 * HID)),
            wfull((EDF, 2 * HID)), wfull((1, 2 * HID)),
            wfull((HID, HID)), wfull((1, HID)),
            wfull((HID, HID)), wfull((1, HID)),
            wfull((1, HID)), wfull((1, HID)), wfull((1, 1)),
        ],
        out_specs=pl.BlockSpec((_BE, D), lambda i: (i, 0)),
        out_shape=jax.ShapeDtypeStruct((E, D), jnp.float32),
        compiler_params=pltpu.CompilerParams(
            dimension_semantics=("arbitrary",)),
    )(gathered, gathered, a, W1s, W1d, w1r, W1a, b1, Wc2, bc2, We2, be2,
      wc3, wa, ba2)


_BN = 2000
_NNB = N // _BN


def _node_mlp_body(h_ref, c16_ref, a0_ref, a1_ref, Wn1h, Wn1g, bn1r, Wn2r,
                   bn2r, hout_ref, cout_ref):
    hagg = a0_ref[0]
    xagg = a1_ref[0][:, :EDF]
    n1 = _silu(jnp.dot(h_ref[:], Wn1h[:], precision=_PREC,
                       preferred_element_type=jnp.float32)
               + jnp.dot(hagg, Wn1g[:], precision=_PREC,
                         preferred_element_type=jnp.float32) + bn1r[:])
    n2 = jnp.dot(n1, Wn2r[:], precision=_PREC,
                 preferred_element_type=jnp.float32) + bn2r[:]
    hout_ref[:] = h_ref[:] + n2
    cout_ref[:] = c16_ref[:] + xagg


def _node_mlp(h, coords16, agg, Wn1h, Wn1g, bn1, Wn2, bn2):
    wfull = lambda shape: pl.BlockSpec(shape, lambda i: (0, 0))
    return pl.pallas_call(
        _node_mlp_body,
        grid=(_NNB,),
        in_specs=[
            pl.BlockSpec((_BN, HID), lambda i: (i, 0)),
            pl.BlockSpec((_BN, EDF), lambda i: (i, 0)),
            pl.BlockSpec((1, _BN, HID), lambda i: (0, i, 0)),
            pl.BlockSpec((1, _BN, HID), lambda i: (1, i, 0)),
            wfull((HID, HID)), wfull((HID, HID)), wfull((1, HID)),
            wfull((HID, HID)), wfull((1, HID)),
        ],
        out_specs=[
            pl.BlockSpec((_BN, HID), lambda i: (i, 0)),
            pl.BlockSpec((_BN, EDF), lambda i: (i, 0)),
        ],
        out_shape=[
            jax.ShapeDtypeStruct((N, HID), jnp.float32),
            jax.ShapeDtypeStruct((N, EDF), jnp.float32),
        ],
        compiler_params=pltpu.CompilerParams(
            dimension_semantics=("arbitrary",)),
    )(h, coords16, agg, agg, Wn1h, Wn1g, bn1, Wn2, bn2)


def kernel(h, coords, edge_index, a, Wc1, bc1, Wc2, bc2, Wc3, We1, be1, We2,
           be2, Wa, ba, Wn1, bn1, Wn2, bn2):
    f32 = jnp.float32
    table = jnp.concatenate(
        [h, coords, jnp.zeros((N, D - HID - 3), f32)], axis=1)
    idx_all = edge_index.reshape(-1).astype(jnp.int32)
    dst = edge_index[1].astype(jnp.int32)

    gathered = _get_gather()(table, idx_all)

    W1s = jnp.concatenate([Wc1[:HID], We1[:HID]], axis=1)
    W1d = jnp.concatenate([Wc1[HID:2 * HID], We1[HID:2 * HID]], axis=1)
    w1r = jnp.concatenate([Wc1[2 * HID:2 * HID + 1],
                           We1[2 * HID:2 * HID + 1]], axis=1)
    W1a = jnp.concatenate([Wc1[2 * HID + 1:], We1[2 * HID + 1:]], axis=1)
    b1 = jnp.concatenate([bc1, be1]).reshape(1, 2 * HID)
    msg = _edge_mlp(gathered, a, W1s, W1d, w1r, W1a, b1, Wc2,
                    bc2.reshape(1, HID), We2, be2.reshape(1, HID),
                    Wc3.reshape(1, HID), Wa.reshape(1, HID),
                    ba.reshape(1, 1))

    z = jnp.zeros((N, HID), f32)
    agg = _get_scatter()(msg, dst, z)

    coords16 = jnp.concatenate(
        [coords, jnp.zeros((N, EDF - 3), f32)], axis=1)
    h_out, cout16 = _node_mlp(h, coords16, agg, Wn1[:HID], Wn1[HID:],
                              bn1.reshape(1, HID), Wn2, bn2.reshape(1, HID))
    return (h_out, cout16[:, :3])


# speedup vs baseline: 2.9262x; 1.6401x over previous
import functools

import jax
import jax.numpy as jnp
from jax import lax
from jax.experimental import pallas as pl
from jax.experimental.pallas import tpu as pltpu
from jax.experimental.pallas import tpu_sc as plsc

N = 10000
E = 320000
HID = 128
EDF = 16
D = 256
NC, NS = 2, 16
NW = NC * NS

_R = 2 * E
_RPW = _R // NW
_GC = 80
_GCHUNKS = _RPW // _GC


def _gather_body(table, idx, out, idx_v, rows_v, sem):
    c = lax.axis_index("c")
    s = lax.axis_index("s")
    wid = s * NC + c
    base = wid * _RPW

    @pl.loop(0, _GCHUNKS)
    def _(i):
        off = base + i * _GC
        pltpu.sync_copy(idx.at[pl.ds(off, _GC)], idx_v)
        pltpu.async_copy(table.at[idx_v], rows_v, sem).wait()
        pltpu.sync_copy(rows_v, out.at[pl.ds(off, _GC)])


@functools.cache
def _get_gather():
    return pl.kernel(
        _gather_body,
        out_type=jax.ShapeDtypeStruct((_R, D), jnp.float32),
        mesh=plsc.VectorSubcoreMesh(core_axis_name="c", subcore_axis_name="s",
                                    num_cores=NC, num_subcores=NS),
        scratch_types=[
            pltpu.VMEM((_GC,), jnp.int32),
            pltpu.VMEM((_GC, D), jnp.float32),
            pltpu.SemaphoreType.DMA,
        ],
    )


_EPT = E // NS
_SCC = 80
_SCHUNKS = _EPT // _SCC
_ZROWS = 632
_ZLAST = N - (NS - 1) * _ZROWS


def _scatter_body(msg, dst, z, out, idx_v, rows_v, acc, sem):
    c = lax.axis_index("c")
    s = lax.axis_index("s")
    r0 = s * _ZROWS

    @pl.when(s < NS - 1)
    def _():
        pltpu.sync_copy(z.at[pl.ds(r0, _ZROWS)], acc.at[pl.ds(r0, _ZROWS)])

    @pl.when(s == NS - 1)
    def _():
        pltpu.sync_copy(z.at[pl.ds(r0, _ZLAST)], acc.at[pl.ds(r0, _ZLAST)])

    plsc.subcore_barrier()

    base = s * _EPT
    col = c * HID

    @pl.loop(0, _SCHUNKS)
    def _(i):
        off = base + i * _SCC
        pltpu.sync_copy(dst.at[pl.ds(off, _SCC)], idx_v)
        pltpu.sync_copy(msg.at[pl.ds(off, _SCC), pl.ds(col, HID)], rows_v)
        pltpu.sync_copy(rows_v, acc.at[idx_v], add=True)

    plsc.subcore_barrier()

    @pl.when(s < NS - 1)
    def _():
        pltpu.sync_copy(acc.at[pl.ds(r0, _ZROWS)],
                        out.at[c, pl.ds(r0, _ZROWS)])

    @pl.when(s == NS - 1)
    def _():
        pltpu.sync_copy(acc.at[pl.ds(r0, _ZLAST)],
                        out.at[c, pl.ds(r0, _ZLAST)])


@functools.cache
def _get_scatter():
    return pl.kernel(
        _scatter_body,
        out_type=jax.ShapeDtypeStruct((NC, N, HID), jnp.float32),
        mesh=plsc.VectorSubcoreMesh(core_axis_name="c", subcore_axis_name="s",
                                    num_cores=NC, num_subcores=NS),
        scratch_types=[
            pltpu.VMEM((_SCC,), jnp.int32),
            pltpu.VMEM((_SCC, HID), jnp.float32),
            pltpu.VMEM_SHARED((N, HID), jnp.float32),
            pltpu.SemaphoreType.DMA,
        ],
    )


_BE = 1280
_NBLK = E // _BE
_PREC = jax.lax.Precision.DEFAULT


def _silu(x):
    return x * jax.nn.sigmoid(x)


def _edge_mlp_body(hs_ref, hd_ref, a_ref, W1s, W1d, w1r, W1a, b1,
                   Wc2r, bc2r, We2r, be2r, wc3, wa, ba2, out_ref):
    hs = hs_ref[:, :HID]
    hd = hd_ref[:, :HID]
    d16 = hs_ref[:, HID:HID + EDF] - hd_ref[:, HID:HID + EDF]
    rad = jnp.sum(d16 * d16, axis=1, keepdims=True)
    t = (jnp.dot(hs, W1s[:], precision=_PREC, preferred_element_type=jnp.float32)
         + jnp.dot(hd, W1d[:], precision=_PREC, preferred_element_type=jnp.float32)
         + jnp.dot(a_ref[:], W1a[:], precision=_PREC, preferred_element_type=jnp.float32)
         + rad * w1r[:] + b1[:])
    t = _silu(t)
    c2 = _silu(jnp.dot(t[:, :HID], Wc2r[:], precision=_PREC,
                       preferred_element_type=jnp.float32) + bc2r[:])
    m2 = _silu(jnp.dot(t[:, HID:], We2r[:], precision=_PREC,
                       preferred_element_type=jnp.float32) + be2r[:])
    scale = jnp.sum(c2 * wc3[:], axis=1, keepdims=True)
    att = jax.nn.sigmoid(jnp.sum(m2 * wa[:], axis=1, keepdims=True) + ba2[:])
    out_ref[:, :HID] = att * m2
    out_ref[:, HID:] = jnp.concatenate(
        [(scale / (rad + 1.0)) * d16,
         jnp.zeros((_BE, D - HID - EDF), jnp.float32)], axis=1)


def _edge_mlp(gathered, a, W1s, W1d, w1r, W1a, b1, Wc2, bc2, We2, be2,
              wc3, wa, ba2):
    wfull = lambda shape: pl.BlockSpec(shape, lambda i: (0, 0))
    return pl.pallas_call(
        _edge_mlp_body,
        grid=(_NBLK,),
        in_specs=[
            pl.BlockSpec((_BE, D), lambda i: (i, 0)),
            pl.BlockSpec((_BE, D), lambda i: (i + _NBLK, 0)),
            pl.BlockSpec((_BE, EDF), lambda i: (i, 0)),
            wfull((HID, 2 * HID)), wfull((HID, 2 * HID)), wfull((1, 2 * HID)),
            wfull((EDF, 2 * HID)), wfull((1, 2 * HID)),
            wfull((HID, HID)), wfull((1, HID)),
            wfull((HID, HID)), wfull((1, HID)),
            wfull((1, HID)), wfull((1, HID)), wfull((1, 1)),
        ],
        out_specs=pl.BlockSpec((_BE, D), lambda i: (i, 0)),
        out_shape=jax.ShapeDtypeStruct((E, D), jnp.float32),
        compiler_params=pltpu.CompilerParams(
            dimension_semantics=("arbitrary",)),
    )(gathered, gathered, a, W1s, W1d, w1r, W1a, b1, Wc2, bc2, We2, be2,
      wc3, wa, ba2)


_BN = 2000
_NNB = N // _BN


def _node_mlp_body(h_ref, c16_ref, a0_ref, a1_ref, Wn1h, Wn1g, bn1r, Wn2r,
                   bn2r, hout_ref, cout_ref):
    hagg = a0_ref[0]
    xagg = a1_ref[0][:, :EDF]
    n1 = _silu(jnp.dot(h_ref[:], Wn1h[:], precision=_PREC,
                       preferred_element_type=jnp.float32)
               + jnp.dot(hagg, Wn1g[:], precision=_PREC,
                         preferred_element_type=jnp.float32) + bn1r[:])
    n2 = jnp.dot(n1, Wn2r[:], precision=_PREC,
                 preferred_element_type=jnp.float32) + bn2r[:]
    hout_ref[:] = h_ref[:] + n2
    cout_ref[:] = c16_ref[:] + xagg


def _node_mlp(h, coords16, agg, Wn1h, Wn1g, bn1, Wn2, bn2):
    wfull = lambda shape: pl.BlockSpec(shape, lambda i: (0, 0))
    return pl.pallas_call(
        _node_mlp_body,
        grid=(_NNB,),
        in_specs=[
            pl.BlockSpec((_BN, HID), lambda i: (i, 0)),
            pl.BlockSpec((_BN, EDF), lambda i: (i, 0)),
            pl.BlockSpec((1, _BN, HID), lambda i: (0, i, 0)),
            pl.BlockSpec((1, _BN, HID), lambda i: (1, i, 0)),
            wfull((HID, HID)), wfull((HID, HID)), wfull((1, HID)),
            wfull((HID, HID)), wfull((1, HID)),
        ],
        out_specs=[
            pl.BlockSpec((_BN, HID), lambda i: (i, 0)),
            pl.BlockSpec((_BN, EDF), lambda i: (i, 0)),
        ],
        out_shape=[
            jax.ShapeDtypeStruct((N, HID), jnp.float32),
            jax.ShapeDtypeStruct((N, EDF), jnp.float32),
        ],
        compiler_params=pltpu.CompilerParams(
            dimension_semantics=("arbitrary",)),
    )(h, coords16, agg, agg, Wn1h, Wn1g, bn1, Wn2, bn2)


def kernel(h, coords, edge_index, a, Wc1, bc1, Wc2, bc2, Wc3, We1, be1, We2,
           be2, Wa, ba, Wn1, bn1, Wn2, bn2):
    f32 = jnp.float32
    table = jnp.concatenate(
        [h, coords, jnp.zeros((N, D - HID - 3), f32)], axis=1)
    idx_all = edge_index.reshape(-1).astype(jnp.int32)
    dst = edge_index[1].astype(jnp.int32)

    gathered = _get_gather()(table, idx_all)

    W1s = jnp.concatenate([Wc1[:HID], We1[:HID]], axis=1)
    W1d = jnp.concatenate([Wc1[HID:2 * HID], We1[HID:2 * HID]], axis=1)
    w1r = jnp.concatenate([Wc1[2 * HID:2 * HID + 1],
                           We1[2 * HID:2 * HID + 1]], axis=1)
    W1a = jnp.concatenate([Wc1[2 * HID + 1:], We1[2 * HID + 1:]], axis=1)
    b1 = jnp.concatenate([bc1, be1]).reshape(1, 2 * HID)
    msg = _edge_mlp(gathered, a, W1s, W1d, w1r, W1a, b1, Wc2,
                    bc2.reshape(1, HID), We2, be2.reshape(1, HID),
                    Wc3.reshape(1, HID), Wa.reshape(1, HID),
                    ba.reshape(1, 1))

    z = jnp.zeros((N, HID), f32)
    agg = _get_scatter()(msg, dst, z)

    coords16 = jnp.concatenate(
        [coords, jnp.zeros((N, EDF - 3), f32)], axis=1)
    h_out, cout16 = _node_mlp(h, coords16, agg, Wn1[:HID], Wn1[HID:],
                              bn1.reshape(1, HID), Wn2, bn2.reshape(1, HID))
    return (h_out, cout16[:, :3])


# speedup vs baseline: 4.0407x; 1.3809x over previous
import functools

import jax
import jax.numpy as jnp
from jax import lax
from jax.experimental import pallas as pl
from jax.experimental.pallas import tpu as pltpu
from jax.experimental.pallas import tpu_sc as plsc

N = 10000
E = 320000
HID = 128
EDF = 16
NC, NS = 2, 16
NW = NC * NS
L = 16

_NP = 10112
_C = 128
_NCH = E // _C
_CPW = _NCH // NW
_NEXTRA = _NCH - _CPW * NW


def _gather_body(h, idx, ct1, out, dT, is_v, id_v, hs_b, hd_b, db, xv, yv,
                 zv, sem):
    c = lax.axis_index("c")
    s = lax.axis_index("s")
    wid = s * NC + c

    pltpu.sync_copy(ct1.at[pl.ds(0, _NP)], xv)
    pltpu.sync_copy(ct1.at[pl.ds(_NP, _NP)], yv)
    pltpu.sync_copy(ct1.at[pl.ds(2 * _NP, _NP)], zv)

    def process(ci):
        off = pl.multiple_of(ci * _C, _C)
        pltpu.sync_copy(idx.at[pl.ds(off, _C)], is_v)
        pltpu.sync_copy(idx.at[pl.ds(E + off, _C)], id_v)
        pltpu.async_copy(h.at[is_v], hs_b, sem).wait()
        pltpu.async_copy(h.at[id_v], hd_b, sem).wait()
        pltpu.sync_copy(hs_b, out.at[pl.ds(off, _C)])
        pltpu.sync_copy(hd_b, out.at[pl.ds(E + off, _C)])
        for j in range(_C // L):
            ivs = is_v[pl.ds(j * L, L)]
            ivd = id_v[pl.ds(j * L, L)]
            dx = plsc.load_gather(xv, [ivs]) - plsc.load_gather(xv, [ivd])
            dy = plsc.load_gather(yv, [ivs]) - plsc.load_gather(yv, [ivd])
            dz = plsc.load_gather(zv, [ivs]) - plsc.load_gather(zv, [ivd])
            rad = dx * dx + dy * dy + dz * dz
            db[0, pl.ds(j * L, L)] = dx
            db[1, pl.ds(j * L, L)] = dy
            db[2, pl.ds(j * L, L)] = dz
            db[3, pl.ds(j * L, L)] = rad
        pltpu.sync_copy(db, dT.at[:, pl.ds(off, _C)])

    @pl.loop(0, _CPW)
    def _(i):
        process(wid + i * NW)

    @pl.when(wid < _NEXTRA)
    def _():
        process(_CPW * NW + wid)


@functools.cache
def _get_gather():
    return pl.kernel(
        _gather_body,
        out_type=(
            jax.ShapeDtypeStruct((2 * E, HID), jnp.float32),
            jax.ShapeDtypeStruct((8, E), jnp.float32),
        ),
        mesh=plsc.VectorSubcoreMesh(core_axis_name="c", subcore_axis_name="s",
                                    num_cores=NC, num_subcores=NS),
        scratch_types=[
            pltpu.VMEM((_C,), jnp.int32),
            pltpu.VMEM((_C,), jnp.int32),
            pltpu.VMEM((_C, HID), jnp.float32),
            pltpu.VMEM((_C, HID), jnp.float32),
            pltpu.VMEM((8, _C), jnp.float32),
            pltpu.VMEM((_NP,), jnp.float32),
            pltpu.VMEM((_NP,), jnp.float32),
            pltpu.VMEM((_NP,), jnp.float32),
            pltpu.SemaphoreType.DMA,
        ],
        compiler_params=pltpu.CompilerParams(needs_layout_passes=False),
    )


_ZROWS = 632
_ZLAST = N - (NS - 1) * _ZROWS
_XR = 320
_CH = 64


def _scatter_body(msg_h, msg_xT, dst, z, out_h, out_x, idx_a, idx_b, rows_v,
                  xb, xacc, acc, sem):
    c = lax.axis_index("c")
    s = lax.axis_index("s")
    wid = s * NC + c
    r0 = s * _ZROWS

    @pl.when(s < NS - 1)
    def _():
        pltpu.sync_copy(z.at[pl.ds(r0, _ZROWS)], acc.at[pl.ds(r0, _ZROWS)])

    @pl.when(s == NS - 1)
    def _():
        pltpu.sync_copy(z.at[pl.ds(r0, _ZLAST)], acc.at[pl.ds(r0, _ZLAST)])

    pltpu.sync_copy(z.at[pl.ds(0, _XR)], xacc)
    plsc.subcore_barrier()

    def process(ci):
        off = pl.multiple_of(ci * _C, _C)
        pltpu.sync_copy(dst.at[pl.ds(off, _CH)], idx_a)
        pltpu.sync_copy(dst.at[pl.ds(off + _CH, _CH)], idx_b)
        pltpu.sync_copy(msg_h.at[pl.ds(off, _CH)], rows_v)
        pltpu.sync_copy(rows_v, acc.at[idx_a], add=True)
        pltpu.sync_copy(msg_h.at[pl.ds(off + _CH, _CH)], rows_v)
        pltpu.sync_copy(rows_v, acc.at[idx_b], add=True)
        pltpu.sync_copy(msg_xT.at[:, pl.ds(off, _C)], xb)
        for j in range(_C // L):
            half = idx_a if j < (_CH // L) else idx_b
            iv = half[pl.ds((j * L) % _CH, L)]
            rowv = jax.lax.shift_right_logical(iv, 5)
            colv = jax.lax.shift_left(iv & 31, 2)
            for k in range(3):
                v = xb[k, pl.ds(j * L, L)]
                plsc.addupdate_scatter(xacc, [rowv, colv + k], v)

    @pl.loop(0, _CPW)
    def _(i):
        process(wid + i * NW)

    @pl.when(wid < _NEXTRA)
    def _():
        process(_CPW * NW + wid)

    plsc.subcore_barrier()

    @pl.when(s < NS - 1)
    def _():
        pltpu.sync_copy(acc.at[pl.ds(r0, _ZROWS)],
                        out_h.at[c, pl.ds(r0, _ZROWS)])

    @pl.when(s == NS - 1)
    def _():
        pltpu.sync_copy(acc.at[pl.ds(r0, _ZLAST)],
                        out_h.at[c, pl.ds(r0, _ZLAST)])

    pltpu.sync_copy(xacc, out_x.at[wid])


@functools.cache
def _get_scatter():
    return pl.kernel(
        _scatter_body,
        out_type=(
            jax.ShapeDtypeStruct((NC, N, HID), jnp.float32),
            jax.ShapeDtypeStruct((NW, _XR, HID), jnp.float32),
        ),
        mesh=plsc.VectorSubcoreMesh(core_axis_name="c", subcore_axis_name="s",
                                    num_cores=NC, num_subcores=NS),
        scratch_types=[
            pltpu.VMEM((_CH,), jnp.int32),
            pltpu.VMEM((_CH,), jnp.int32),
            pltpu.VMEM((_CH, HID), jnp.float32),
            pltpu.VMEM((8, _C), jnp.float32),
            pltpu.VMEM((_XR, HID), jnp.float32),
            pltpu.VMEM_SHARED((N, HID), jnp.float32),
            pltpu.SemaphoreType.DMA,
        ],
        compiler_params=pltpu.CompilerParams(needs_layout_passes=False),
    )


_BE = 1280
_NBLK = E // _BE
_PREC = jax.lax.Precision.DEFAULT
_DN = (((0,), (0,)), ((), ()))
_DN1 = (((1,), (1,)), ((), ()))


def _silu(x):
    return x * jax.nn.sigmoid(x)


def _edge_mlp_body(hs_ref, hd_ref, dT_ref, a_ref, W1s, W1d, w1r, W1a, b1,
                   Wc2r, bc2r, We2r, be2r, wc3, wa_c, ba2, outh_ref,
                   outx_ref):
    rad = dT_ref[3:4, :]
    t = (jnp.dot(hs_ref[:], W1s[:], precision=_PREC,
                 preferred_element_type=jnp.float32)
         + jnp.dot(hd_ref[:], W1d[:], precision=_PREC,
                   preferred_element_type=jnp.float32)
         + jnp.dot(a_ref[:], W1a[:], precision=_PREC,
                   preferred_element_type=jnp.float32)
         + lax.dot_general(rad, w1r[:], _DN, precision=_PREC,
                           preferred_element_type=jnp.float32)
         + b1[:])
    t = _silu(t)
    c2 = _silu(jnp.dot(t[:, :HID], Wc2r[:], precision=_PREC,
                       preferred_element_type=jnp.float32) + bc2r[:])
    m2 = _silu(jnp.dot(t[:, HID:], We2r[:], precision=_PREC,
                       preferred_element_type=jnp.float32) + be2r[:])
    scale = lax.dot_general(wc3[:], c2, _DN1, precision=_PREC,
                            preferred_element_type=jnp.float32)
    att = jax.nn.sigmoid(jnp.dot(m2, wa_c[:], precision=_PREC,
                                 preferred_element_type=jnp.float32)
                         + ba2[:])
    outh_ref[:] = att * m2
    w_row = scale / (rad + 1.0)
    outx_ref[:] = jnp.concatenate(
        [w_row * dT_ref[0:3, :], jnp.zeros((5, _BE), jnp.float32)], axis=0)


def _edge_mlp(gathered, dT, a, W1s, W1d, w1r, W1a, b1, Wc2, bc2, We2, be2,
              wc3, wa_c, ba2):
    wfull = lambda shape: pl.BlockSpec(shape, lambda i: (0, 0))
    return pl.pallas_call(
        _edge_mlp_body,
        grid=(_NBLK,),
        in_specs=[
            pl.BlockSpec((_BE, HID), lambda i: (i, 0)),
            pl.BlockSpec((_BE, HID), lambda i: (i + _NBLK, 0)),
            pl.BlockSpec((8, _BE), lambda i: (0, i)),
            pl.BlockSpec((_BE, EDF), lambda i: (i, 0)),
            wfull((HID, 2 * HID)), wfull((HID, 2 * HID)), wfull((1, 2 * HID)),
            wfull((EDF, 2 * HID)), wfull((1, 2 * HID)),
            wfull((HID, HID)), wfull((1, HID)),
            wfull((HID, HID)), wfull((1, HID)),
            wfull((1, HID)), wfull((HID, 1)), wfull((1, 1)),
        ],
        out_specs=[
            pl.BlockSpec((_BE, HID), lambda i: (i, 0)),
            pl.BlockSpec((8, _BE), lambda i: (0, i)),
        ],
        out_shape=[
            jax.ShapeDtypeStruct((E, HID), jnp.float32),
            jax.ShapeDtypeStruct((8, E), jnp.float32),
        ],
        compiler_params=pltpu.CompilerParams(
            dimension_semantics=("arbitrary",)),
    )(gathered, gathered, dT, a, W1s, W1d, w1r, W1a, b1, Wc2, bc2, We2, be2,
      wc3, wa_c, ba2)


_BN = 2000
_NNB = N // _BN


def _node_mlp_body(h_ref, a0_ref, a1_ref, Wn1h, Wn1g, bn1r, Wn2r, bn2r,
                   hout_ref):
    hagg = a0_ref[0] + a1_ref[0]
    n1 = _silu(jnp.dot(h_ref[:], Wn1h[:], precision=_PREC,
                       preferred_element_type=jnp.float32)
               + jnp.dot(hagg, Wn1g[:], precision=_PREC,
                         preferred_element_type=jnp.float32) + bn1r[:])
    n2 = jnp.dot(n1, Wn2r[:], precision=_PREC,
                 preferred_element_type=jnp.float32) + bn2r[:]
    hout_ref[:] = h_ref[:] + n2


def _node_mlp(h, agg, Wn1h, Wn1g, bn1, Wn2, bn2):
    wfull = lambda shape: pl.BlockSpec(shape, lambda i: (0, 0))
    return pl.pallas_call(
        _node_mlp_body,
        grid=(_NNB,),
        in_specs=[
            pl.BlockSpec((_BN, HID), lambda i: (i, 0)),
            pl.BlockSpec((1, _BN, HID), lambda i: (0, i, 0)),
            pl.BlockSpec((1, _BN, HID), lambda i: (1, i, 0)),
            wfull((HID, HID)), wfull((HID, HID)), wfull((1, HID)),
            wfull((HID, HID)), wfull((1, HID)),
        ],
        out_specs=pl.BlockSpec((_BN, HID), lambda i: (i, 0)),
        out_shape=jax.ShapeDtypeStruct((N, HID), jnp.float32),
        compiler_params=pltpu.CompilerParams(
            dimension_semantics=("arbitrary",)),
    )(h, agg, agg, Wn1h, Wn1g, bn1, Wn2, bn2)


def _coords_body(cp_ref, x_ref, cout_ref):
    xs = x_ref[0]
    for p in range(1, NW):
        xs = xs + x_ref[p]
    cout_ref[:] = cp_ref[:] + xs


def _coords_out(cpack, out_x):
    return pl.pallas_call(
        _coords_body,
        grid=(1,),
        in_specs=[
            pl.BlockSpec((_XR, HID), lambda i: (0, 0)),
            pl.BlockSpec((NW, _XR, HID), lambda i: (0, 0, 0)),
        ],
        out_specs=pl.BlockSpec((_XR, HID), lambda i: (0, 0)),
        out_shape=jax.ShapeDtypeStruct((_XR, HID), jnp.float32),
    )(cpack, out_x)


def kernel(h, coords, edge_index, a, Wc1, bc1, Wc2, bc2, Wc3, We1, be1, We2,
           be2, Wa, ba, Wn1, bn1, Wn2, bn2):
    f32 = jnp.float32
    idx_all = edge_index.reshape(-1).astype(jnp.int32)
    dst = edge_index[1].astype(jnp.int32)
    ct1 = jnp.pad(coords.T, ((0, 0), (0, _NP - N))).reshape(-1)

    gathered, dT = _get_gather()(h, idx_all, ct1)

    W1s = jnp.concatenate([Wc1[:HID], We1[:HID]], axis=1)
    W1d = jnp.concatenate([Wc1[HID:2 * HID], We1[HID:2 * HID]], axis=1)
    w1r = jnp.concatenate([Wc1[2 * HID:2 * HID + 1],
                           We1[2 * HID:2 * HID + 1]], axis=1)
    W1a = jnp.concatenate([Wc1[2 * HID + 1:], We1[2 * HID + 1:]], axis=1)
    b1 = jnp.concatenate([bc1, be1]).reshape(1, 2 * HID)
    msg_h, msg_xT = _edge_mlp(gathered, dT, a, W1s, W1d, w1r, W1a, b1, Wc2,
                              bc2.reshape(1, HID), We2, be2.reshape(1, HID),
                              Wc3.reshape(1, HID), Wa, ba.reshape(1, 1))

    z = jnp.zeros((N, HID), f32)
    agg, out_x = _get_scatter()(msg_h, msg_xT, dst, z)

    h_out = _node_mlp(h, agg, Wn1[:HID], Wn1[HID:],
                      bn1.reshape(1, HID), Wn2, bn2.reshape(1, HID))
    cpack = jnp.pad(coords, ((0, _XR * 32 - N), (0, 1))).reshape(_XR, HID)
    cout = _coords_out(cpack, out_x)
    coords_out = cout.reshape(_XR * 32, 4)[:N, :3]
    return (h_out, coords_out)


# speedup vs baseline: 4.4541x; 1.1023x over previous
import functools

import jax
import jax.numpy as jnp
from jax import lax
from jax.experimental import pallas as pl
from jax.experimental.pallas import tpu as pltpu
from jax.experimental.pallas import tpu_sc as plsc

N = 10000
E = 320000
HID = 128
EDF = 16
NC, NS = 2, 16
NW = NC * NS
L = 16

_NP = 10112
_C = 128
_NCH = E // _C
_CPW = _NCH // NW
_NEXTRA = _NCH - _CPW * NW


def _gather_body(h, idx, ct1, out, dT, is_v, id_v, hs_b, hd_b, db, xv, yv,
                 zv, sem):
    c = lax.axis_index("c")
    s = lax.axis_index("s")
    wid = s * NC + c

    pltpu.sync_copy(ct1.at[pl.ds(0, _NP)], xv)
    pltpu.sync_copy(ct1.at[pl.ds(_NP, _NP)], yv)
    pltpu.sync_copy(ct1.at[pl.ds(2 * _NP, _NP)], zv)

    def process(ci):
        off = pl.multiple_of(ci * _C, _C)
        pltpu.sync_copy(idx.at[pl.ds(off, _C)], is_v)
        pltpu.sync_copy(idx.at[pl.ds(E + off, _C)], id_v)
        pltpu.async_copy(h.at[is_v], hs_b, sem).wait()
        pltpu.async_copy(h.at[id_v], hd_b, sem).wait()
        pltpu.sync_copy(hs_b, out.at[pl.ds(off, _C), pl.ds(0, HID)])
        pltpu.sync_copy(hd_b, out.at[pl.ds(off, _C), pl.ds(HID, HID)])
        for j in range(_C // L):
            ivs = is_v[pl.ds(j * L, L)]
            ivd = id_v[pl.ds(j * L, L)]
            dx = plsc.load_gather(xv, [ivs]) - plsc.load_gather(xv, [ivd])
            dy = plsc.load_gather(yv, [ivs]) - plsc.load_gather(yv, [ivd])
            dz = plsc.load_gather(zv, [ivs]) - plsc.load_gather(zv, [ivd])
            rad = dx * dx + dy * dy + dz * dz
            db[0, pl.ds(j * L, L)] = dx
            db[1, pl.ds(j * L, L)] = dy
            db[2, pl.ds(j * L, L)] = dz
            db[3, pl.ds(j * L, L)] = rad
        pltpu.sync_copy(db, dT.at[:, pl.ds(off, _C)])

    @pl.loop(0, _CPW)
    def _(i):
        process(wid + i * NW)

    @pl.when(wid < _NEXTRA)
    def _():
        process(_CPW * NW + wid)


@functools.cache
def _get_gather():
    return pl.kernel(
        _gather_body,
        out_type=(
            jax.ShapeDtypeStruct((E, 2 * HID), jnp.float32),
            jax.ShapeDtypeStruct((8, E), jnp.float32),
        ),
        mesh=plsc.VectorSubcoreMesh(core_axis_name="c", subcore_axis_name="s",
                                    num_cores=NC, num_subcores=NS),
        scratch_types=[
            pltpu.VMEM((_C,), jnp.int32),
            pltpu.VMEM((_C,), jnp.int32),
            pltpu.VMEM((_C, HID), jnp.float32),
            pltpu.VMEM((_C, HID), jnp.float32),
            pltpu.VMEM((8, _C), jnp.float32),
            pltpu.VMEM((_NP,), jnp.float32),
            pltpu.VMEM((_NP,), jnp.float32),
            pltpu.VMEM((_NP,), jnp.float32),
            pltpu.SemaphoreType.DMA,
        ],
        compiler_params=pltpu.CompilerParams(needs_layout_passes=False),
    )


_ZROWS = 632
_ZLAST = N - (NS - 1) * _ZROWS
_XR = 320
_CH = 64


def _scatter_body(msg_h, msg_xT, dst, z, out_h, out_x, idx_a, idx_b, rows_v,
                  xb, xacc, acc, sem):
    c = lax.axis_index("c")
    s = lax.axis_index("s")
    wid = s * NC + c
    r0 = s * _ZROWS

    @pl.when(s < NS - 1)
    def _():
        pltpu.sync_copy(z.at[pl.ds(r0, _ZROWS)], acc.at[pl.ds(r0, _ZROWS)])

    @pl.when(s == NS - 1)
    def _():
        pltpu.sync_copy(z.at[pl.ds(r0, _ZLAST)], acc.at[pl.ds(r0, _ZLAST)])

    pltpu.sync_copy(z.at[pl.ds(0, _XR)], xacc)
    plsc.subcore_barrier()

    def process(ci):
        off = pl.multiple_of(ci * _C, _C)
        pltpu.sync_copy(dst.at[pl.ds(off, _CH)], idx_a)
        pltpu.sync_copy(dst.at[pl.ds(off + _CH, _CH)], idx_b)
        pltpu.sync_copy(msg_h.at[pl.ds(off, _CH)], rows_v)
        pltpu.sync_copy(rows_v, acc.at[idx_a], add=True)
        pltpu.sync_copy(msg_h.at[pl.ds(off + _CH, _CH)], rows_v)
        pltpu.sync_copy(rows_v, acc.at[idx_b], add=True)
        pltpu.sync_copy(msg_xT.at[:, pl.ds(off, _C)], xb)
        for j in range(_C // L):
            half = idx_a if j < (_CH // L) else idx_b
            iv = half[pl.ds((j * L) % _CH, L)]
            rowv = jax.lax.shift_right_logical(iv, 5)
            colv = jax.lax.shift_left(iv & 31, 2)
            for k in range(3):
                v = xb[k, pl.ds(j * L, L)]
                plsc.addupdate_scatter(xacc, [rowv, colv + k], v)

    @pl.loop(0, _CPW)
    def _(i):
        process(wid + i * NW)

    @pl.when(wid < _NEXTRA)
    def _():
        process(_CPW * NW + wid)

    plsc.subcore_barrier()

    @pl.when(s < NS - 1)
    def _():
        pltpu.sync_copy(acc.at[pl.ds(r0, _ZROWS)],
                        out_h.at[c, pl.ds(r0, _ZROWS)])

    @pl.when(s == NS - 1)
    def _():
        pltpu.sync_copy(acc.at[pl.ds(r0, _ZLAST)],
                        out_h.at[c, pl.ds(r0, _ZLAST)])

    pltpu.sync_copy(xacc, out_x.at[wid])


@functools.cache
def _get_scatter():
    return pl.kernel(
        _scatter_body,
        out_type=(
            jax.ShapeDtypeStruct((NC, N, HID), jnp.float32),
            jax.ShapeDtypeStruct((NW, _XR, HID), jnp.float32),
        ),
        mesh=plsc.VectorSubcoreMesh(core_axis_name="c", subcore_axis_name="s",
                                    num_cores=NC, num_subcores=NS),
        scratch_types=[
            pltpu.VMEM((_CH,), jnp.int32),
            pltpu.VMEM((_CH,), jnp.int32),
            pltpu.VMEM((_CH, HID), jnp.float32),
            pltpu.VMEM((8, _C), jnp.float32),
            pltpu.VMEM((_XR, HID), jnp.float32),
            pltpu.VMEM_SHARED((N, HID), jnp.float32),
            pltpu.SemaphoreType.DMA,
        ],
        compiler_params=pltpu.CompilerParams(needs_layout_passes=False),
    )


_BE = 2560
_NBLK = E // _BE
_PREC = jax.lax.Precision.DEFAULT
_DN = (((0,), (0,)), ((), ()))
_DN1 = (((1,), (1,)), ((), ()))


def _silu(x):
    return x * jax.nn.sigmoid(x)


def _edge_mlp_body(hsd_ref, dT_ref, a_ref, W1sd, w1r, W1a, b1,
                   Wc2r, bc2r, We2r, be2r, wc3, wa_c, ba2, outh_ref,
                   outx_ref):
    bf16 = jnp.bfloat16
    rad = dT_ref[3:4, :]
    t = (jnp.dot(hsd_ref[:], W1sd[:], precision=_PREC,
                 preferred_element_type=jnp.float32)
         + jnp.dot(a_ref[:], W1a[:], precision=_PREC,
                   preferred_element_type=jnp.float32)
         + lax.dot_general(rad, w1r[:], _DN, precision=_PREC,
                           preferred_element_type=jnp.float32)
         + b1[:])
    t = _silu(t.astype(bf16))
    c2 = _silu((jnp.dot(t[:, :HID], Wc2r[:].astype(bf16), precision=_PREC,
                        preferred_element_type=jnp.float32)
                + bc2r[:]).astype(bf16))
    m2 = _silu((jnp.dot(t[:, HID:], We2r[:].astype(bf16), precision=_PREC,
                        preferred_element_type=jnp.float32)
                + be2r[:]).astype(bf16))
    scale = lax.dot_general(wc3[:].astype(bf16), c2, _DN1, precision=_PREC,
                            preferred_element_type=jnp.float32)
    att = jax.nn.sigmoid(jnp.dot(m2, wa_c[:].astype(bf16), precision=_PREC,
                                 preferred_element_type=jnp.float32)
                         + ba2[:])
    outh_ref[:] = att * m2.astype(jnp.float32)
    w_row = scale / (rad + 1.0)
    outx_ref[:] = jnp.concatenate(
        [w_row * dT_ref[0:3, :], jnp.zeros((5, _BE), jnp.float32)], axis=0)


def _edge_mlp(gathered, dT, a, W1sd, w1r, W1a, b1, Wc2, bc2, We2, be2,
              wc3, wa_c, ba2):
    wfull = lambda shape: pl.BlockSpec(shape, lambda i: (0, 0))
    return pl.pallas_call(
        _edge_mlp_body,
        grid=(_NBLK,),
        in_specs=[
            pl.BlockSpec((_BE, 2 * HID), lambda i: (i, 0)),
            pl.BlockSpec((8, _BE), lambda i: (0, i)),
            pl.BlockSpec((_BE, EDF), lambda i: (i, 0)),
            wfull((2 * HID, 2 * HID)), wfull((1, 2 * HID)),
            wfull((EDF, 2 * HID)), wfull((1, 2 * HID)),
            wfull((HID, HID)), wfull((1, HID)),
            wfull((HID, HID)), wfull((1, HID)),
            wfull((1, HID)), wfull((HID, 1)), wfull((1, 1)),
        ],
        out_specs=[
            pl.BlockSpec((_BE, HID), lambda i: (i, 0)),
            pl.BlockSpec((8, _BE), lambda i: (0, i)),
        ],
        out_shape=[
            jax.ShapeDtypeStruct((E, HID), jnp.float32),
            jax.ShapeDtypeStruct((8, E), jnp.float32),
        ],
        compiler_params=pltpu.CompilerParams(
            dimension_semantics=("parallel",)),
    )(gathered, dT, a, W1sd, w1r, W1a, b1, Wc2, bc2, We2, be2,
      wc3, wa_c, ba2)


_BN = 2000
_NNB = N // _BN


def _node_mlp_body(h_ref, a0_ref, a1_ref, Wn1h, Wn1g, bn1r, Wn2r, bn2r,
                   hout_ref):
    hagg = a0_ref[0] + a1_ref[0]
    n1 = _silu(jnp.dot(h_ref[:], Wn1h[:], precision=_PREC,
                       preferred_element_type=jnp.float32)
               + jnp.dot(hagg, Wn1g[:], precision=_PREC,
                         preferred_element_type=jnp.float32) + bn1r[:])
    n2 = jnp.dot(n1, Wn2r[:], precision=_PREC,
                 preferred_element_type=jnp.float32) + bn2r[:]
    hout_ref[:] = h_ref[:] + n2


def _node_mlp(h, agg, Wn1h, Wn1g, bn1, Wn2, bn2):
    wfull = lambda shape: pl.BlockSpec(shape, lambda i: (0, 0))
    return pl.pallas_call(
        _node_mlp_body,
        grid=(_NNB,),
        in_specs=[
            pl.BlockSpec((_BN, HID), lambda i: (i, 0)),
            pl.BlockSpec((1, _BN, HID), lambda i: (0, i, 0)),
            pl.BlockSpec((1, _BN, HID), lambda i: (1, i, 0)),
            wfull((HID, HID)), wfull((HID, HID)), wfull((1, HID)),
            wfull((HID, HID)), wfull((1, HID)),
        ],
        out_specs=pl.BlockSpec((_BN, HID), lambda i: (i, 0)),
        out_shape=jax.ShapeDtypeStruct((N, HID), jnp.float32),
        compiler_params=pltpu.CompilerParams(
            dimension_semantics=("arbitrary",)),
    )(h, agg, agg, Wn1h, Wn1g, bn1, Wn2, bn2)


def _coords_body(cp_ref, x_ref, cout_ref):
    xs = x_ref[0]
    for p in range(1, NW):
        xs = xs + x_ref[p]
    cout_ref[:] = cp_ref[:] + xs


def _coords_out(cpack, out_x):
    return pl.pallas_call(
        _coords_body,
        grid=(1,),
        in_specs=[
            pl.BlockSpec((_XR, HID), lambda i: (0, 0)),
            pl.BlockSpec((NW, _XR, HID), lambda i: (0, 0, 0)),
        ],
        out_specs=pl.BlockSpec((_XR, HID), lambda i: (0, 0)),
        out_shape=jax.ShapeDtypeStruct((_XR, HID), jnp.float32),
    )(cpack, out_x)


def kernel(h, coords, edge_index, a, Wc1, bc1, Wc2, bc2, Wc3, We1, be1, We2,
           be2, Wa, ba, Wn1, bn1, Wn2, bn2):
    f32 = jnp.float32
    idx_all = edge_index.reshape(-1).astype(jnp.int32)
    dst = edge_index[1].astype(jnp.int32)
    ct1 = jnp.pad(coords.T, ((0, 0), (0, _NP - N))).reshape(-1)

    gathered, dT = _get_gather()(h, idx_all, ct1)

    W1sd = jnp.concatenate([Wc1[:2 * HID], We1[:2 * HID]], axis=1)
    w1r = jnp.concatenate([Wc1[2 * HID:2 * HID + 1],
                           We1[2 * HID:2 * HID + 1]], axis=1)
    W1a = jnp.concatenate([Wc1[2 * HID + 1:], We1[2 * HID + 1:]], axis=1)
    b1 = jnp.concatenate([bc1, be1]).reshape(1, 2 * HID)
    msg_h, msg_xT = _edge_mlp(gathered, dT, a, W1sd, w1r, W1a, b1, Wc2,
                              bc2.reshape(1, HID), We2, be2.reshape(1, HID),
                              Wc3.reshape(1, HID), Wa, ba.reshape(1, 1))

    z = jnp.zeros((N, HID), f32)
    agg, out_x = _get_scatter()(msg_h, msg_xT, dst, z)

    h_out = _node_mlp(h, agg, Wn1[:HID], Wn1[HID:],
                      bn1.reshape(1, HID), Wn2, bn2.reshape(1, HID))
    cpack = jnp.pad(coords, ((0, _XR * 32 - N), (0, 1))).reshape(_XR, HID)
    cout = _coords_out(cpack, out_x)
    coords_out = cout.reshape(_XR * 32, 4)[:N, :3]
    return (h_out, coords_out)


# speedup vs baseline: 5.6463x; 1.2677x over previous
import functools

import jax
import jax.numpy as jnp
from jax import lax
from jax.experimental import pallas as pl
from jax.experimental.pallas import tpu as pltpu
from jax.experimental.pallas import tpu_sc as plsc

N = 10000
E = 320000
HID = 128
EDF = 16
NC, NS = 2, 16
NW = NC * NS
L = 16

_NP = 10112
_C = 128
_EH = E // 2
_NCH = _EH // _C
_CPW = _NCH // NW
_NEXTRA = _NCH - _CPW * NW


def _make_gather_body(eoff):
    def _gather_body(h, idx, ct1, out, dT, is_v, id_v, hs_b, hd_b, db, xv,
                     yv, zv, sem):
        c = lax.axis_index("c")
        s = lax.axis_index("s")
        wid = s * NC + c

        pltpu.sync_copy(ct1.at[pl.ds(0, _NP)], xv)
        pltpu.sync_copy(ct1.at[pl.ds(_NP, _NP)], yv)
        pltpu.sync_copy(ct1.at[pl.ds(2 * _NP, _NP)], zv)

        def process(ci):
            off = pl.multiple_of(ci * _C, _C)
            pltpu.sync_copy(idx.at[pl.ds(eoff + off, _C)], is_v)
            pltpu.sync_copy(idx.at[pl.ds(E + eoff + off, _C)], id_v)
            pltpu.async_copy(h.at[is_v], hs_b, sem).wait()
            pltpu.async_copy(h.at[id_v], hd_b, sem).wait()
            pltpu.sync_copy(hs_b, out.at[pl.ds(off, _C), pl.ds(0, HID)])
            pltpu.sync_copy(hd_b, out.at[pl.ds(off, _C), pl.ds(HID, HID)])
            for j in range(_C // L):
                ivs = is_v[pl.ds(j * L, L)]
                ivd = id_v[pl.ds(j * L, L)]
                dx = plsc.load_gather(xv, [ivs]) - plsc.load_gather(xv, [ivd])
                dy = plsc.load_gather(yv, [ivs]) - plsc.load_gather(yv, [ivd])
                dz = plsc.load_gather(zv, [ivs]) - plsc.load_gather(zv, [ivd])
                rad = dx * dx + dy * dy + dz * dz
                db[0, pl.ds(j * L, L)] = dx
                db[1, pl.ds(j * L, L)] = dy
                db[2, pl.ds(j * L, L)] = dz
                db[3, pl.ds(j * L, L)] = rad
            pltpu.sync_copy(db, dT.at[:, pl.ds(off, _C)])

        @pl.loop(0, _CPW)
        def _(i):
            process(wid + i * NW)

        @pl.when(wid < _NEXTRA)
        def _():
            process(_CPW * NW + wid)

    return _gather_body


@functools.cache
def _get_gather(eoff):
    return pl.kernel(
        _make_gather_body(eoff),
        out_type=(
            jax.ShapeDtypeStruct((_EH, 2 * HID), jnp.float32),
            jax.ShapeDtypeStruct((8, _EH), jnp.float32),
        ),
        mesh=plsc.VectorSubcoreMesh(core_axis_name="c", subcore_axis_name="s",
                                    num_cores=NC, num_subcores=NS),
        scratch_types=[
            pltpu.VMEM((_C,), jnp.int32),
            pltpu.VMEM((_C,), jnp.int32),
            pltpu.VMEM((_C, HID), jnp.float32),
            pltpu.VMEM((_C, HID), jnp.float32),
            pltpu.VMEM((8, _C), jnp.float32),
            pltpu.VMEM((_NP,), jnp.float32),
            pltpu.VMEM((_NP,), jnp.float32),
            pltpu.VMEM((_NP,), jnp.float32),
            pltpu.SemaphoreType.DMA,
        ],
        compiler_params=pltpu.CompilerParams(needs_layout_passes=False),
    )


_ZROWS = 632
_ZLAST = N - (NS - 1) * _ZROWS
_XR = 320
_CH = 64


def _make_scatter_body(eoff):
    def _scatter_body(msg_h, msg_xT, dst, z, out_h, out_x, idx_a, idx_b,
                      rows_v, xb, xacc, acc, sem):
        c = lax.axis_index("c")
        s = lax.axis_index("s")
        wid = s * NC + c
        r0 = s * _ZROWS

        @pl.when(s < NS - 1)
        def _():
            pltpu.sync_copy(z.at[pl.ds(r0, _ZROWS)],
                            acc.at[pl.ds(r0, _ZROWS)])

        @pl.when(s == NS - 1)
        def _():
            pltpu.sync_copy(z.at[pl.ds(r0, _ZLAST)],
                            acc.at[pl.ds(r0, _ZLAST)])

        pltpu.sync_copy(z.at[pl.ds(0, _XR)], xacc)
        plsc.subcore_barrier()

        def process(ci):
            off = pl.multiple_of(ci * _C, _C)
            pltpu.sync_copy(dst.at[pl.ds(eoff + off, _CH)], idx_a)
            pltpu.sync_copy(dst.at[pl.ds(eoff + off + _CH, _CH)], idx_b)
            pltpu.sync_copy(msg_h.at[pl.ds(off, _CH)], rows_v)
            pltpu.sync_copy(rows_v, acc.at[idx_a], add=True)
            pltpu.sync_copy(msg_h.at[pl.ds(off + _CH, _CH)], rows_v)
            pltpu.sync_copy(rows_v, acc.at[idx_b], add=True)
            pltpu.sync_copy(msg_xT.at[:, pl.ds(off, _C)], xb)
            for j in range(_C // L):
                half = idx_a if j < (_CH // L) else idx_b
                iv = half[pl.ds((j * L) % _CH, L)]
                rowv = jax.lax.shift_right_logical(iv, 5)
                colv = jax.lax.shift_left(iv & 31, 2)
                for k in range(3):
                    v = xb[k, pl.ds(j * L, L)]
                    plsc.addupdate_scatter(xacc, [rowv, colv + k], v)

        @pl.loop(0, _CPW)
        def _(i):
            process(wid + i * NW)

        @pl.when(wid < _NEXTRA)
        def _():
            process(_CPW * NW + wid)

        plsc.subcore_barrier()

        @pl.when(s < NS - 1)
        def _():
            pltpu.sync_copy(acc.at[pl.ds(r0, _ZROWS)],
                            out_h.at[c, pl.ds(r0, _ZROWS)])

        @pl.when(s == NS - 1)
        def _():
            pltpu.sync_copy(acc.at[pl.ds(r0, _ZLAST)],
                            out_h.at[c, pl.ds(r0, _ZLAST)])

        pltpu.sync_copy(xacc, out_x.at[wid])

    return _scatter_body


@functools.cache
def _get_scatter(eoff):
    return pl.kernel(
        _make_scatter_body(eoff),
        out_type=(
            jax.ShapeDtypeStruct((NC, N, HID), jnp.float32),
            jax.ShapeDtypeStruct((NW, _XR, HID), jnp.float32),
        ),
        mesh=plsc.VectorSubcoreMesh(core_axis_name="c", subcore_axis_name="s",
                                    num_cores=NC, num_subcores=NS),
        scratch_types=[
            pltpu.VMEM((_CH,), jnp.int32),
            pltpu.VMEM((_CH,), jnp.int32),
            pltpu.VMEM((_CH, HID), jnp.float32),
            pltpu.VMEM((8, _C), jnp.float32),
            pltpu.VMEM((_XR, HID), jnp.float32),
            pltpu.VMEM_SHARED((N, HID), jnp.float32),
            pltpu.SemaphoreType.DMA,
        ],
        compiler_params=pltpu.CompilerParams(needs_layout_passes=False),
    )


_BE = 3200
_NBLK = _EH // _BE
_PREC = jax.lax.Precision.DEFAULT
_DN = (((0,), (0,)), ((), ()))
_DN1 = (((1,), (1,)), ((), ()))


def _silu(x):
    return x * jax.nn.sigmoid(x)


def _edge_mlp_body(hsd_ref, dT_ref, a_ref, W1sd, w1r, W1a, b1,
                   Wc2r, bc2r, We2r, be2r, wc3, wa_c, ba2, outh_ref,
                   outx_ref):
    bf16 = jnp.bfloat16
    rad = dT_ref[3:4, :]
    t = (jnp.dot(hsd_ref[:], W1sd[:], precision=_PREC,
                 preferred_element_type=jnp.float32)
         + jnp.dot(a_ref[:], W1a[:], precision=_PREC,
                   preferred_element_type=jnp.float32)
         + lax.dot_general(rad, w1r[:], _DN, precision=_PREC,
                           preferred_element_type=jnp.float32)
         + b1[:])
    t = _silu(t.astype(bf16))
    c2 = _silu((jnp.dot(t[:, :HID], Wc2r[:].astype(bf16), precision=_PREC,
                        preferred_element_type=jnp.float32)
                + bc2r[:]).astype(bf16))
    m2 = _silu((jnp.dot(t[:, HID:], We2r[:].astype(bf16), precision=_PREC,
                        preferred_element_type=jnp.float32)
                + be2r[:]).astype(bf16))
    scale = lax.dot_general(wc3[:].astype(bf16), c2, _DN1, precision=_PREC,
                            preferred_element_type=jnp.float32)
    att = jax.nn.sigmoid(jnp.dot(m2, wa_c[:].astype(bf16), precision=_PREC,
                                 preferred_element_type=jnp.float32)
                         + ba2[:])
    outh_ref[:] = att * m2.astype(jnp.float32)
    w_row = scale / (rad + 1.0)
    outx_ref[:] = jnp.concatenate(
        [w_row * dT_ref[0:3, :], jnp.zeros((5, _BE), jnp.float32)], axis=0)


def _edge_mlp(gathered, dT, a, W1sd, w1r, W1a, b1, Wc2, bc2, We2, be2,
              wc3, wa_c, ba2, boff):
    wfull = lambda shape: pl.BlockSpec(shape, lambda i: (0, 0))
    return pl.pallas_call(
        _edge_mlp_body,
        grid=(_NBLK,),
        in_specs=[
            pl.BlockSpec((_BE, 2 * HID), lambda i: (i, 0)),
            pl.BlockSpec((8, _BE), lambda i: (0, i)),
            pl.BlockSpec((_BE, EDF), lambda i, b=boff: (i + b, 0)),
            wfull((2 * HID, 2 * HID)), wfull((1, 2 * HID)),
            wfull((EDF, 2 * HID)), wfull((1, 2 * HID)),
            wfull((HID, HID)), wfull((1, HID)),
            wfull((HID, HID)), wfull((1, HID)),
            wfull((1, HID)), wfull((HID, 1)), wfull((1, 1)),
        ],
        out_specs=[
            pl.BlockSpec((_BE, HID), lambda i: (i, 0)),
            pl.BlockSpec((8, _BE), lambda i: (0, i)),
        ],
        out_shape=[
            jax.ShapeDtypeStruct((_EH, HID), jnp.float32),
            jax.ShapeDtypeStruct((8, _EH), jnp.float32),
        ],
        compiler_params=pltpu.CompilerParams(
            dimension_semantics=("parallel",)),
    )(gathered, dT, a, W1sd, w1r, W1a, b1, Wc2, bc2, We2, be2,
      wc3, wa_c, ba2)


_BN = 2000
_NNB = N // _BN


def _node_mlp_body(h_ref, a0_ref, a1_ref, a2_ref, a3_ref, Wn1h, Wn1g, bn1r,
                   Wn2r, bn2r, hout_ref):
    hagg = a0_ref[0] + a1_ref[0] + a2_ref[0] + a3_ref[0]
    n1 = _silu(jnp.dot(h_ref[:], Wn1h[:], precision=_PREC,
                       preferred_element_type=jnp.float32)
               + jnp.dot(hagg, Wn1g[:], precision=_PREC,
                         preferred_element_type=jnp.float32) + bn1r[:])
    n2 = jnp.dot(n1, Wn2r[:], precision=_PREC,
                 preferred_element_type=jnp.float32) + bn2r[:]
    hout_ref[:] = h_ref[:] + n2


def _node_mlp(h, agg1, agg2, Wn1h, Wn1g, bn1, Wn2, bn2):
    wfull = lambda shape: pl.BlockSpec(shape, lambda i: (0, 0))
    return pl.pallas_call(
        _node_mlp_body,
        grid=(_NNB,),
        in_specs=[
            pl.BlockSpec((_BN, HID), lambda i: (i, 0)),
            pl.BlockSpec((1, _BN, HID), lambda i: (0, i, 0)),
            pl.BlockSpec((1, _BN, HID), lambda i: (1, i, 0)),
            pl.BlockSpec((1, _BN, HID), lambda i: (0, i, 0)),
            pl.BlockSpec((1, _BN, HID), lambda i: (1, i, 0)),
            wfull((HID, HID)), wfull((HID, HID)), wfull((1, HID)),
            wfull((HID, HID)), wfull((1, HID)),
        ],
        out_specs=pl.BlockSpec((_BN, HID), lambda i: (i, 0)),
        out_shape=jax.ShapeDtypeStruct((N, HID), jnp.float32),
        compiler_params=pltpu.CompilerParams(
            dimension_semantics=("arbitrary",)),
    )(h, agg1, agg1, agg2, agg2, Wn1h, Wn1g, bn1, Wn2, bn2)


def _coords_body(cp_ref, x1_ref, x2_ref, cout_ref):
    xs = cp_ref[:]
    for p in range(NW):
        xs = xs + x1_ref[p] + x2_ref[p]
    cout_ref[:] = xs


def _coords_out(cpack, out_x1, out_x2):
    return pl.pallas_call(
        _coords_body,
        grid=(1,),
        in_specs=[
            pl.BlockSpec((_XR, HID), lambda i: (0, 0)),
            pl.BlockSpec((NW, _XR, HID), lambda i: (0, 0, 0)),
            pl.BlockSpec((NW, _XR, HID), lambda i: (0, 0, 0)),
        ],
        out_specs=pl.BlockSpec((_XR, HID), lambda i: (0, 0)),
        out_shape=jax.ShapeDtypeStruct((_XR, HID), jnp.float32),
    )(cpack, out_x1, out_x2)


def kernel(h, coords, edge_index, a, Wc1, bc1, Wc2, bc2, Wc3, We1, be1, We2,
           be2, Wa, ba, Wn1, bn1, Wn2, bn2):
    f32 = jnp.float32
    idx_all = edge_index.reshape(-1).astype(jnp.int32)
    dst = edge_index[1].astype(jnp.int32)
    ct1 = jnp.pad(coords.T, ((0, 0), (0, _NP - N))).reshape(-1)

    W1sd = jnp.concatenate([Wc1[:2 * HID], We1[:2 * HID]], axis=1)
    w1r = jnp.concatenate([Wc1[2 * HID:2 * HID + 1],
                           We1[2 * HID:2 * HID + 1]], axis=1)
    W1a = jnp.concatenate([Wc1[2 * HID + 1:], We1[2 * HID + 1:]], axis=1)
    b1 = jnp.concatenate([bc1, be1]).reshape(1, 2 * HID)
    z = jnp.zeros((N, HID), f32)

    aggs, oxs = [], []
    for half in range(2):
        gathered, dT = _get_gather(half * _EH)(h, idx_all, ct1)
        msg_h, msg_xT = _edge_mlp(gathered, dT, a, W1sd, w1r, W1a, b1, Wc2,
                                  bc2.reshape(1, HID), We2,
                                  be2.reshape(1, HID), Wc3.reshape(1, HID),
                                  Wa, ba.reshape(1, 1), half * _NBLK)
        agg, out_x = _get_scatter(half * _EH)(msg_h, msg_xT, dst, z)
        aggs.append(agg)
        oxs.append(out_x)

    h_out = _node_mlp(h, aggs[0], aggs[1], Wn1[:HID], Wn1[HID:],
                      bn1.reshape(1, HID), Wn2, bn2.reshape(1, HID))
    cpack = jnp.pad(coords, ((0, _XR * 32 - N), (0, 1))).reshape(_XR, HID)
    cout = _coords_out(cpack, oxs[0], oxs[1])
    coords_out = cout.reshape(_XR * 32, 4)[:N, :3]
    return (h_out, coords_out)


# speedup vs baseline: 6.1750x; 1.0936x over previous
import functools

import jax
import jax.numpy as jnp
from jax import lax
from jax.experimental import pallas as pl
from jax.experimental.pallas import tpu as pltpu
from jax.experimental.pallas import tpu_sc as plsc

N = 10000
E = 320000
HID = 128
EDF = 16
NC, NS = 2, 16
NW = NC * NS
L = 16

_NP = 10112
_C = 128
_EH = E // 2
_NCH = _EH // _C
_CPW = _NCH // NW
_NEXTRA = _NCH - _CPW * NW


_CPIPE = _CPW + 1


def _make_gather_body(eoff):
    def _gather_body(h, idx, ct1, out, dT, is_v0, id_v0, hs_b0, hd_b0, db0,
                     is_v1, id_v1, hs_b1, hd_b1, db1, xv, yv, zv, sem0,
                     sem1):
        c = lax.axis_index("c")
        s = lax.axis_index("s")
        wid = s * NC + c

        pltpu.sync_copy(ct1.at[pl.ds(0, _NP)], xv)
        pltpu.sync_copy(ct1.at[pl.ds(_NP, _NP)], yv)
        pltpu.sync_copy(ct1.at[pl.ds(2 * _NP, _NP)], zv)

        sets = ((is_v0, id_v0, hs_b0, hd_b0, db0, sem0),
                (is_v1, id_v1, hs_b1, hd_b1, db1, sem1))

        def chunk_of(i):
            base = wid + i * NW
            alt = jnp.where(wid < _NEXTRA, _CPW * NW + wid, wid)
            return jnp.where(i == _CPW, alt, base)

        def issue(i, st):
            is_v, id_v, hs_b, hd_b, db, sem = st
            off = pl.multiple_of(chunk_of(i) * _C, _C)
            pltpu.sync_copy(idx.at[pl.ds(eoff + off, _C)], is_v)
            pltpu.sync_copy(idx.at[pl.ds(E + eoff + off, _C)], id_v)
            pltpu.async_copy(h.at[is_v], hs_b, sem)
            pltpu.async_copy(h.at[id_v], hd_b, sem)

        def finish(i, st):
            is_v, id_v, hs_b, hd_b, db, sem = st
            off = pl.multiple_of(chunk_of(i) * _C, _C)
            pltpu.make_async_copy(h.at[is_v], hs_b, sem).wait()
            pltpu.make_async_copy(h.at[id_v], hd_b, sem).wait()
            pltpu.sync_copy(hs_b, out.at[pl.ds(off, _C), pl.ds(0, HID)])
            pltpu.sync_copy(hd_b, out.at[pl.ds(off, _C), pl.ds(HID, HID)])
            for j in range(_C // L):
                ivs = is_v[pl.ds(j * L, L)]
                ivd = id_v[pl.ds(j * L, L)]
                dx = plsc.load_gather(xv, [ivs]) - plsc.load_gather(xv, [ivd])
                dy = plsc.load_gather(yv, [ivs]) - plsc.load_gather(yv, [ivd])
                dz = plsc.load_gather(zv, [ivs]) - plsc.load_gather(zv, [ivd])
                rad = dx * dx + dy * dy + dz * dz
                db[0, pl.ds(j * L, L)] = dx
                db[1, pl.ds(j * L, L)] = dy
                db[2, pl.ds(j * L, L)] = dz
                db[3, pl.ds(j * L, L)] = rad
            pltpu.sync_copy(db, dT.at[:, pl.ds(off, _C)])

        issue(0, sets[0])

        @pl.loop(0, _CPIPE // 2)
        def _(j):
            i0 = 2 * j
            issue(i0 + 1, sets[1])
            finish(i0, sets[0])

            @pl.when(i0 + 2 < _CPIPE)
            def _():
                issue(i0 + 2, sets[0])

            finish(i0 + 1, sets[1])

    return _gather_body


@functools.cache
def _get_gather(eoff):
    return pl.kernel(
        _make_gather_body(eoff),
        out_type=(
            jax.ShapeDtypeStruct((_EH, 2 * HID), jnp.float32),
            jax.ShapeDtypeStruct((8, _EH), jnp.float32),
        ),
        mesh=plsc.VectorSubcoreMesh(core_axis_name="c", subcore_axis_name="s",
                                    num_cores=NC, num_subcores=NS),
        scratch_types=[
            pltpu.VMEM((_C,), jnp.int32),
            pltpu.VMEM((_C,), jnp.int32),
            pltpu.VMEM((_C, HID), jnp.float32),
            pltpu.VMEM((_C, HID), jnp.float32),
            pltpu.VMEM((8, _C), jnp.float32),
            pltpu.VMEM((_C,), jnp.int32),
            pltpu.VMEM((_C,), jnp.int32),
            pltpu.VMEM((_C, HID), jnp.float32),
            pltpu.VMEM((_C, HID), jnp.float32),
            pltpu.VMEM((8, _C), jnp.float32),
            pltpu.VMEM((_NP,), jnp.float32),
            pltpu.VMEM((_NP,), jnp.float32),
            pltpu.VMEM((_NP,), jnp.float32),
            pltpu.SemaphoreType.DMA,
            pltpu.SemaphoreType.DMA,
        ],
        compiler_params=pltpu.CompilerParams(needs_layout_passes=False),
    )


_ZROWS = 632
_ZLAST = N - (NS - 1) * _ZROWS
_XR = 320
_CH = 64


def _make_scatter_body(eoff):
    def _scatter_body(msg_h, msg_xT, dst, z, out_h, out_x, idx_a, idx_b,
                      rows_v, xb, xacc, acc, sem):
        c = lax.axis_index("c")
        s = lax.axis_index("s")
        wid = s * NC + c
        r0 = s * _ZROWS

        @pl.when(s < NS - 1)
        def _():
            pltpu.sync_copy(z.at[pl.ds(r0, _ZROWS)],
                            acc.at[pl.ds(r0, _ZROWS)])

        @pl.when(s == NS - 1)
        def _():
            pltpu.sync_copy(z.at[pl.ds(r0, _ZLAST)],
                            acc.at[pl.ds(r0, _ZLAST)])

        pltpu.sync_copy(z.at[pl.ds(0, _XR)], xacc)
        plsc.subcore_barrier()

        def process(ci):
            off = pl.multiple_of(ci * _C, _C)
            pltpu.sync_copy(dst.at[pl.ds(eoff + off, _CH)], idx_a)
            pltpu.sync_copy(dst.at[pl.ds(eoff + off + _CH, _CH)], idx_b)
            pltpu.sync_copy(msg_h.at[pl.ds(off, _CH)], rows_v)
            pltpu.sync_copy(rows_v, acc.at[idx_a], add=True)
            pltpu.sync_copy(msg_h.at[pl.ds(off + _CH, _CH)], rows_v)
            pltpu.sync_copy(rows_v, acc.at[idx_b], add=True)
            pltpu.sync_copy(msg_xT.at[:, pl.ds(off, _C)], xb)
            for j in range(_C // L):
                half = idx_a if j < (_CH // L) else idx_b
                iv = half[pl.ds((j * L) % _CH, L)]
                rowv = jax.lax.shift_right_logical(iv, 5)
                colv = jax.lax.shift_left(iv & 31, 2)
                for k in range(3):
                    v = xb[k, pl.ds(j * L, L)]
                    plsc.addupdate_scatter(xacc, [rowv, colv + k], v)

        @pl.loop(0, _CPW)
        def _(i):
            process(wid + i * NW)

        @pl.when(wid < _NEXTRA)
        def _():
            process(_CPW * NW + wid)

        plsc.subcore_barrier()

        @pl.when(s < NS - 1)
        def _():
            pltpu.sync_copy(acc.at[pl.ds(r0, _ZROWS)],
                            out_h.at[c, pl.ds(r0, _ZROWS)])

        @pl.when(s == NS - 1)
        def _():
            pltpu.sync_copy(acc.at[pl.ds(r0, _ZLAST)],
                            out_h.at[c, pl.ds(r0, _ZLAST)])

        pltpu.sync_copy(xacc, out_x.at[wid])

    return _scatter_body


@functools.cache
def _get_scatter(eoff):
    return pl.kernel(
        _make_scatter_body(eoff),
        out_type=(
            jax.ShapeDtypeStruct((NC, N, HID), jnp.float32),
            jax.ShapeDtypeStruct((NW, _XR, HID), jnp.float32),
        ),
        mesh=plsc.VectorSubcoreMesh(core_axis_name="c", subcore_axis_name="s",
                                    num_cores=NC, num_subcores=NS),
        scratch_types=[
            pltpu.VMEM((_CH,), jnp.int32),
            pltpu.VMEM((_CH,), jnp.int32),
            pltpu.VMEM((_CH, HID), jnp.float32),
            pltpu.VMEM((8, _C), jnp.float32),
            pltpu.VMEM((_XR, HID), jnp.float32),
            pltpu.VMEM_SHARED((N, HID), jnp.float32),
            pltpu.SemaphoreType.DMA,
        ],
        compiler_params=pltpu.CompilerParams(needs_layout_passes=False),
    )


_BE = 3200
_NBLK = _EH // _BE
_PREC = jax.lax.Precision.DEFAULT
_DN = (((0,), (0,)), ((), ()))
_DN1 = (((1,), (1,)), ((), ()))


def _silu(x):
    return x * jax.nn.sigmoid(x)


def _edge_mlp_body(hsd_ref, dT_ref, a_ref, W1sd, w1r, W1a, b1,
                   Wc2r, bc2r, We2r, be2r, wc3, wa_c, ba2, outh_ref,
                   outx_ref):
    bf16 = jnp.bfloat16
    rad = dT_ref[3:4, :]
    t = (jnp.dot(hsd_ref[:], W1sd[:], precision=_PREC,
                 preferred_element_type=jnp.float32)
         + jnp.dot(a_ref[:], W1a[:], precision=_PREC,
                   preferred_element_type=jnp.float32)
         + lax.dot_general(rad, w1r[:], _DN, precision=_PREC,
                           preferred_element_type=jnp.float32)
         + b1[:])
    t = _silu(t.astype(bf16))
    c2 = _silu((jnp.dot(t[:, :HID], Wc2r[:].astype(bf16), precision=_PREC,
                        preferred_element_type=jnp.float32)
                + bc2r[:]).astype(bf16))
    m2 = _silu((jnp.dot(t[:, HID:], We2r[:].astype(bf16), precision=_PREC,
                        preferred_element_type=jnp.float32)
                + be2r[:]).astype(bf16))
    scale = lax.dot_general(wc3[:].astype(bf16), c2, _DN1, precision=_PREC,
                            preferred_element_type=jnp.float32)
    att = jax.nn.sigmoid(jnp.dot(m2, wa_c[:].astype(bf16), precision=_PREC,
                                 preferred_element_type=jnp.float32)
                         + ba2[:])
    outh_ref[:] = att * m2.astype(jnp.float32)
    w_row = scale / (rad + 1.0)
    outx_ref[:] = jnp.concatenate(
        [w_row * dT_ref[0:3, :], jnp.zeros((5, _BE), jnp.float32)], axis=0)


def _edge_mlp(gathered, dT, a, W1sd, w1r, W1a, b1, Wc2, bc2, We2, be2,
              wc3, wa_c, ba2, boff):
    wfull = lambda shape: pl.BlockSpec(shape, lambda i: (0, 0))
    return pl.pallas_call(
        _edge_mlp_body,
        grid=(_NBLK,),
        in_specs=[
            pl.BlockSpec((_BE, 2 * HID), lambda i: (i, 0)),
            pl.BlockSpec((8, _BE), lambda i: (0, i)),
            pl.BlockSpec((_BE, EDF), lambda i, b=boff: (i + b, 0)),
            wfull((2 * HID, 2 * HID)), wfull((1, 2 * HID)),
            wfull((EDF, 2 * HID)), wfull((1, 2 * HID)),
            wfull((HID, HID)), wfull((1, HID)),
            wfull((HID, HID)), wfull((1, HID)),
            wfull((1, HID)), wfull((HID, 1)), wfull((1, 1)),
        ],
        out_specs=[
            pl.BlockSpec((_BE, HID), lambda i: (i, 0)),
            pl.BlockSpec((8, _BE), lambda i: (0, i)),
        ],
        out_shape=[
            jax.ShapeDtypeStruct((_EH, HID), jnp.float32),
            jax.ShapeDtypeStruct((8, _EH), jnp.float32),
        ],
        compiler_params=pltpu.CompilerParams(
            dimension_semantics=("parallel",)),
    )(gathered, dT, a, W1sd, w1r, W1a, b1, Wc2, bc2, We2, be2,
      wc3, wa_c, ba2)


_BN = 2000
_NNB = N // _BN


def _node_mlp_body(h_ref, a0_ref, a1_ref, a2_ref, a3_ref, Wn1h, Wn1g, bn1r,
                   Wn2r, bn2r, hout_ref):
    hagg = a0_ref[0] + a1_ref[0] + a2_ref[0] + a3_ref[0]
    n1 = _silu(jnp.dot(h_ref[:], Wn1h[:], precision=_PREC,
                       preferred_element_type=jnp.float32)
               + jnp.dot(hagg, Wn1g[:], precision=_PREC,
                         preferred_element_type=jnp.float32) + bn1r[:])
    n2 = jnp.dot(n1, Wn2r[:], precision=_PREC,
                 preferred_element_type=jnp.float32) + bn2r[:]
    hout_ref[:] = h_ref[:] + n2


def _node_mlp(h, agg1, agg2, Wn1h, Wn1g, bn1, Wn2, bn2):
    wfull = lambda shape: pl.BlockSpec(shape, lambda i: (0, 0))
    return pl.pallas_call(
        _node_mlp_body,
        grid=(_NNB,),
        in_specs=[
            pl.BlockSpec((_BN, HID), lambda i: (i, 0)),
            pl.BlockSpec((1, _BN, HID), lambda i: (0, i, 0)),
            pl.BlockSpec((1, _BN, HID), lambda i: (1, i, 0)),
            pl.BlockSpec((1, _BN, HID), lambda i: (0, i, 0)),
            pl.BlockSpec((1, _BN, HID), lambda i: (1, i, 0)),
            wfull((HID, HID)), wfull((HID, HID)), wfull((1, HID)),
            wfull((HID, HID)), wfull((1, HID)),
        ],
        out_specs=pl.BlockSpec((_BN, HID), lambda i: (i, 0)),
        out_shape=jax.ShapeDtypeStruct((N, HID), jnp.float32),
        compiler_params=pltpu.CompilerParams(
            dimension_semantics=("arbitrary",)),
    )(h, agg1, agg1, agg2, agg2, Wn1h, Wn1g, bn1, Wn2, bn2)


def _coords_body(cp_ref, x1_ref, x2_ref, cout_ref):
    xs = cp_ref[:]
    for p in range(NW):
        xs = xs + x1_ref[p] + x2_ref[p]
    cout_ref[:] = xs


def _coords_out(cpack, out_x1, out_x2):
    return pl.pallas_call(
        _coords_body,
        grid=(1,),
        in_specs=[
            pl.BlockSpec((_XR, HID), lambda i: (0, 0)),
            pl.BlockSpec((NW, _XR, HID), lambda i: (0, 0, 0)),
            pl.BlockSpec((NW, _XR, HID), lambda i: (0, 0, 0)),
        ],
        out_specs=pl.BlockSpec((_XR, HID), lambda i: (0, 0)),
        out_shape=jax.ShapeDtypeStruct((_XR, HID), jnp.float32),
    )(cpack, out_x1, out_x2)


def kernel(h, coords, edge_index, a, Wc1, bc1, Wc2, bc2, Wc3, We1, be1, We2,
           be2, Wa, ba, Wn1, bn1, Wn2, bn2):
    f32 = jnp.float32
    idx_all = edge_index.reshape(-1).astype(jnp.int32)
    dst = edge_index[1].astype(jnp.int32)
    ct1 = jnp.pad(coords.T, ((0, 0), (0, _NP - N))).reshape(-1)

    W1sd = jnp.concatenate([Wc1[:2 * HID], We1[:2 * HID]], axis=1)
    w1r = jnp.concatenate([Wc1[2 * HID:2 * HID + 1],
                           We1[2 * HID:2 * HID + 1]], axis=1)
    W1a = jnp.concatenate([Wc1[2 * HID + 1:], We1[2 * HID + 1:]], axis=1)
    b1 = jnp.concatenate([bc1, be1]).reshape(1, 2 * HID)
    z = jnp.zeros((N, HID), f32)

    aggs, oxs = [], []
    for half in range(2):
        gathered, dT = _get_gather(half * _EH)(h, idx_all, ct1)
        msg_h, msg_xT = _edge_mlp(gathered, dT, a, W1sd, w1r, W1a, b1, Wc2,
                                  bc2.reshape(1, HID), We2,
                                  be2.reshape(1, HID), Wc3.reshape(1, HID),
                                  Wa, ba.reshape(1, 1), half * _NBLK)
        agg, out_x = _get_scatter(half * _EH)(msg_h, msg_xT, dst, z)
        aggs.append(agg)
        oxs.append(out_x)

    h_out = _node_mlp(h, aggs[0], aggs[1], Wn1[:HID], Wn1[HID:],
                      bn1.reshape(1, HID), Wn2, bn2.reshape(1, HID))
    cpack = jnp.pad(coords, ((0, _XR * 32 - N), (0, 1))).reshape(_XR, HID)
    cout = _coords_out(cpack, oxs[0], oxs[1])
    coords_out = cout.reshape(_XR * 32, 4)[:N, :3]
    return (h_out, coords_out)
